# baseline (speedup 1.0000x reference)
import functools

import jax
import jax.numpy as jnp
from jax import lax
from jax.experimental import pallas as pl
from jax.experimental.pallas import tpu as pltpu
from jax.experimental.pallas import tpu_sc as plsc

NQ = 1024
D = 16
NS = 100000
BLK = 2048
NB = 49
NSP = NB * BLK
G = 32
GPB = BLK // G
NG = NSP // G
K = 16
NCAND = K * G
CH = 24
PADV = 1.0e6
BIGF = 3.0e38
BIGI = 2**31 - 1
QBLK3 = 64


def _tc1_body(qT_ref, q2_ref, s_ref, s2_ref, gidx_ref, gmin_ref):
    b = pl.program_id(0)
    qT = qT_ref[...]
    sb = s_ref[...]
    s2c = s2_ref[0]
    q2r = q2_ref[...]
    cross = lax.dot_general(
        sb.astype(jnp.bfloat16), qT.astype(jnp.bfloat16),
        (((1,), (0,)), ((), ())), preferred_element_type=jnp.float32)
    d2 = jnp.maximum((q2r + s2c) - 2.0 * cross, 0.0)
    r = d2.reshape(GPB, G, NQ)
    a = jnp.minimum(r[:, :16, :], r[:, 16:, :])
    a = jnp.minimum(a[:, :8, :], a[:, 8:, :])
    gmin_ref[b] = jnp.min(a, axis=1)

    @pl.when(b == NB - 1)
    def _extract():
        iota0 = lax.broadcasted_iota(jnp.int32, (GPB, NQ), 0)

        def extract_round(t, sel_prev):
            def scan_chunk(cb, inner):
                mv, mi = inner
                ch = gmin_ref[cb]
                cid = iota0 + cb * GPB
                ch = jnp.where(cid == sel_prev, BIGF, ch)
                gmin_ref[cb] = ch
                upd = ch < mv
                return (jnp.where(upd, ch, mv), jnp.where(upd, cid, mi))

            mv, mi = lax.fori_loop(
                0, NB, scan_chunk,
                (jnp.full((GPB, NQ), BIGF, jnp.float32),
                 jnp.full((GPB, NQ), BIGI, jnp.int32)))
            m = jnp.min(mv, axis=0, keepdims=True)
            sel = jnp.min(jnp.where(mv == m, mi, BIGI), axis=0, keepdims=True)
            gidx_ref[pl.ds(t, 1), :] = sel
            return sel

        lax.fori_loop(0, K, extract_round,
                      jnp.full((1, NQ), -1, jnp.int32))


def _tc1_call(qT, q2r, s_pad, s2_3d):
    return pl.pallas_call(
        _tc1_body,
        grid=(NB,),
        in_specs=[
            pl.BlockSpec((D, NQ), lambda b: (0, 0)),
            pl.BlockSpec((1, NQ), lambda b: (0, 0)),
            pl.BlockSpec((BLK, D), lambda b: (b, 0)),
            pl.BlockSpec((1, BLK, 1), lambda b: (b, 0, 0)),
        ],
        out_specs=pl.BlockSpec((K, NQ), lambda b: (0, 0)),
        out_shape=jax.ShapeDtypeStruct((K, NQ), jnp.int32),
        scratch_shapes=[pltpu.VMEM((NB, GPB, NQ), jnp.float32)],
    )(qT, q2r, s_pad, s2_3d)


def _sc_gather(gidx_flat, s_aug3):
    info = plsc.get_sparse_core_info()
    nc, nsub = info.num_cores, info.num_subcores
    nw = nc * nsub
    qpt = NQ // nw

    @functools.partial(
        pl.kernel,
        out_type=[
            jax.ShapeDtypeStruct((NQ, K, (G * CH) // 128, 128), jnp.float32),
        ],
        mesh=plsc.VectorSubcoreMesh(core_axis_name="c", subcore_axis_name="s"),
        scratch_types=[
            pltpu.VMEM((qpt * K,), jnp.int32),
            pltpu.VMEM((K, (G * CH) // 128, 128), jnp.float32),
            pltpu.SemaphoreType.DMA,
        ],
    )
    def _kern(gidx_hbm, saug_hbm, outr_hbm, glv, rowsv, sem):
        wid = lax.axis_index("s") * nc + lax.axis_index("c")
        qbase = wid * qpt
        pltpu.sync_copy(gidx_hbm.at[pl.ds(qbase * K, qpt * K)], glv)

        def body(qi, carry):
            cp = pltpu.async_copy(
                saug_hbm.at[glv.at[pl.ds(qi * K, K)]], rowsv, sem)
            cp.wait()
            pltpu.sync_copy(rowsv, outr_hbm.at[qbase + qi])
            return carry

        lax.fori_loop(0, qpt, body, 0)

    return _kern(gidx_flat, s_aug3)


def _tc3_body(qaT_ref, q2_ref, cr_ref, cs2_ref, ci_ref, vals_ref, idx_ref):
    qa = qaT_ref[0]
    crb = cr_ref[...]
    cib = ci_ref[...]
    cs2b = cs2_ref[...]
    q2c = q2_ref[:, 0:1]
    cross = lax.dot_general(
        crb.astype(jnp.bfloat16), qa.astype(jnp.bfloat16),
        (((1,), (0,)), ((), ())), preferred_element_type=jnp.float32)
    r3 = cross.reshape(QBLK3, NCAND, QBLK3)
    li = lax.broadcasted_iota(jnp.int32, (QBLK3, 1, QBLK3), 0)
    lk = lax.broadcasted_iota(jnp.int32, (QBLK3, 1, QBLK3), 2)
    diag = li == lk
    cd = jnp.sum(jnp.where(diag, r3, 0.0), axis=2)
    d2 = jnp.maximum((q2c + cs2b) - 2.0 * cd, 0.0)
    ds = jnp.sqrt(d2)
    vals = []
    idxs = []
    for _t in range(K):
        m = jnp.min(ds, axis=1, keepdims=True)
        si = jnp.min(jnp.where(ds == m, cib, BIGI), axis=1, keepdims=True)
        ds = jnp.where(cib == si, BIGF, ds)
        vals.append(m)
        idxs.append(si)
    vals_ref[...] = jnp.concatenate(vals, axis=1)
    idx_ref[...] = jnp.concatenate(idxs, axis=1)


def _tc3_call(qaT3, q2b, cr, cs2, ci):
    nsteps = NQ // QBLK3
    return pl.pallas_call(
        _tc3_body,
        grid=(nsteps,),
        in_specs=[
            pl.BlockSpec((1, CH, QBLK3), lambda i: (i, 0, 0)),
            pl.BlockSpec((QBLK3, 128), lambda i: (i, 0)),
            pl.BlockSpec((QBLK3 * NCAND, CH), lambda i: (i, 0)),
            pl.BlockSpec((QBLK3, NCAND), lambda i: (i, 0)),
            pl.BlockSpec((QBLK3, NCAND), lambda i: (i, 0)),
        ],
        out_specs=[
            pl.BlockSpec((QBLK3, K), lambda i: (i, 0)),
            pl.BlockSpec((QBLK3, K), lambda i: (i, 0)),
        ],
        out_shape=[
            jax.ShapeDtypeStruct((NQ, K), jnp.float32),
            jax.ShapeDtypeStruct((NQ, K), jnp.int32),
        ],
    )(qaT3, q2b, cr, cs2, ci)


def kernel(query, support):
    q = query[0]
    s = support[0]
    s_pad = jnp.concatenate(
        [s, jnp.full((NSP - NS, D), PADV, jnp.float32)], axis=0)
    s2 = jnp.sum(s_pad * s_pad, axis=-1)
    q2 = jnp.sum(q * q, axis=-1)
    qT = q.T
    q2r = q2.reshape(1, NQ)
    q2b = jnp.broadcast_to(q2[:, None], (NQ, 128))
    s2_3d = s2.reshape(NB, BLK, 1)
    sidxf = jnp.arange(NSP, dtype=jnp.float32)[:, None]
    s_aug = jnp.concatenate(
        [s_pad, s2[:, None], sidxf,
         jnp.zeros((NSP, CH - D - 2), jnp.float32)],
        axis=1)
    s_aug3 = s_aug.reshape(NG, (G * CH) // 128, 128)
    qaT = jnp.concatenate([qT, jnp.zeros((CH - D, NQ), jnp.float32)], axis=0)
    qaT3 = qaT.reshape(CH, NQ // QBLK3, QBLK3).swapaxes(0, 1)

    gidx_T = _tc1_call(qT, q2r, s_pad, s2_3d)
    gidx_flat = gidx_T.T.reshape(NQ * K)
    (rows,) = _sc_gather(gidx_flat, s_aug3)
    rows4 = rows.reshape(NQ, NCAND, CH)
    cr = rows.reshape(NQ * NCAND, CH)
    cs2 = rows4[:, :, D]
    ci = rows4[:, :, D + 1].astype(jnp.int32)
    vals, idx = _tc3_call(qaT3, q2b, cr, cs2, ci)
    return (vals[None], idx[None])


# speedup vs baseline: 1.1929x; 1.1929x over previous
import functools

import jax
import jax.numpy as jnp
from jax import lax
from jax.experimental import pallas as pl
from jax.experimental.pallas import tpu as pltpu
from jax.experimental.pallas import tpu_sc as plsc

NQ = 1024
D = 16
NS = 100000
BLK = 2048
NB = 49
NSP = NB * BLK
G = 16
GPB = BLK // G
NG = NSP // G
K = 16
NCAND = K * G
CH = 24
PADV = 1.0e6
BIGF = 3.0e38
BIGI = 2**31 - 1
QBLK3 = 64


def _tc1_body(qT_ref, q2_ref, s_ref, s2_ref, gidx_ref, gmin_ref):
    b = pl.program_id(0)
    qT = qT_ref[...]
    sb = s_ref[...]
    s2c = s2_ref[0]
    q2r = q2_ref[...]
    cross = lax.dot_general(
        sb.astype(jnp.bfloat16), qT.astype(jnp.bfloat16),
        (((1,), (0,)), ((), ())), preferred_element_type=jnp.float32)
    d2 = jnp.maximum((q2r + s2c) - 2.0 * cross, 0.0)
    r = d2.reshape(GPB, G, NQ)
    a = jnp.minimum(r[:, :8, :], r[:, 8:, :])
    gmin_ref[b] = jnp.min(a, axis=1)

    @pl.when(b == NB - 1)
    def _extract():
        iota0 = lax.broadcasted_iota(jnp.int32, (GPB, NQ), 0)

        def extract_round(t, sel_prev):
            def scan_chunk(cb, inner):
                mv, mi = inner
                ch = gmin_ref[cb]
                cid = iota0 + cb * GPB
                ch = jnp.where(cid == sel_prev, BIGF, ch)
                gmin_ref[cb] = ch
                upd = ch < mv
                return (jnp.where(upd, ch, mv), jnp.where(upd, cid, mi))

            mv, mi = lax.fori_loop(
                0, NB, scan_chunk,
                (jnp.full((GPB, NQ), BIGF, jnp.float32),
                 jnp.full((GPB, NQ), BIGI, jnp.int32)),
                unroll=7)
            m = jnp.min(mv, axis=0, keepdims=True)
            sel = jnp.min(jnp.where(mv == m, mi, BIGI), axis=0, keepdims=True)
            gidx_ref[pl.ds(t, 1), :] = sel
            return sel

        lax.fori_loop(0, K, extract_round,
                      jnp.full((1, NQ), -1, jnp.int32))


def _tc1_call(qT, q2r, s_pad, s2_3d):
    return pl.pallas_call(
        _tc1_body,
        grid=(NB,),
        in_specs=[
            pl.BlockSpec((D, NQ), lambda b: (0, 0)),
            pl.BlockSpec((1, NQ), lambda b: (0, 0)),
            pl.BlockSpec((BLK, D), lambda b: (b, 0)),
            pl.BlockSpec((1, BLK, 1), lambda b: (b, 0, 0)),
        ],
        out_specs=pl.BlockSpec((K, NQ), lambda b: (0, 0)),
        out_shape=jax.ShapeDtypeStruct((K, NQ), jnp.int32),
        scratch_shapes=[pltpu.VMEM((NB, GPB, NQ), jnp.float32)],
    )(qT, q2r, s_pad, s2_3d)


def _sc_gather(gidx_flat, s_aug3):
    info = plsc.get_sparse_core_info()
    nc, nsub = info.num_cores, info.num_subcores
    nw = nc * nsub
    qpt = NQ // nw

    @functools.partial(
        pl.kernel,
        out_type=[
            jax.ShapeDtypeStruct((NQ, K, (G * CH) // 128, 128), jnp.float32),
        ],
        mesh=plsc.VectorSubcoreMesh(core_axis_name="c", subcore_axis_name="s"),
        scratch_types=[
            pltpu.VMEM((qpt * K,), jnp.int32),
            pltpu.VMEM((K, (G * CH) // 128, 128), jnp.float32),
            pltpu.SemaphoreType.DMA,
        ],
    )
    def _kern(gidx_hbm, saug_hbm, outr_hbm, glv, rowsv, sem):
        wid = lax.axis_index("s") * nc + lax.axis_index("c")
        qbase = wid * qpt
        pltpu.sync_copy(gidx_hbm.at[pl.ds(qbase * K, qpt * K)], glv)

        def body(qi, carry):
            cp = pltpu.async_copy(
                saug_hbm.at[glv.at[pl.ds(qi * K, K)]], rowsv, sem)
            cp.wait()
            pltpu.sync_copy(rowsv, outr_hbm.at[qbase + qi])
            return carry

        lax.fori_loop(0, qpt, body, 0)

    return _kern(gidx_flat, s_aug3)


def _tc3_body(qaT_ref, q2_ref, cr_ref, cs2_ref, ci_ref, vals_ref, idx_ref):
    qa = qaT_ref[0]
    crb = cr_ref[...]
    cib = ci_ref[...]
    cs2b = cs2_ref[...]
    q2c = q2_ref[:, 0:1]
    cross = lax.dot_general(
        crb.astype(jnp.bfloat16), qa.astype(jnp.bfloat16),
        (((1,), (0,)), ((), ())), preferred_element_type=jnp.float32)
    r3 = cross.reshape(QBLK3, NCAND, QBLK3)
    li = lax.broadcasted_iota(jnp.int32, (QBLK3, 1, QBLK3), 0)
    lk = lax.broadcasted_iota(jnp.int32, (QBLK3, 1, QBLK3), 2)
    diag = li == lk
    cd = jnp.sum(jnp.where(diag, r3, 0.0), axis=2)
    d2 = jnp.maximum((q2c + cs2b) - 2.0 * cd, 0.0)
    ds = jnp.sqrt(d2)
    vals = []
    idxs = []
    for _t in range(K):
        m = jnp.min(ds, axis=1, keepdims=True)
        si = jnp.min(jnp.where(ds == m, cib, BIGI), axis=1, keepdims=True)
        ds = jnp.where(cib == si, BIGF, ds)
        vals.append(m)
        idxs.append(si)
    vals_ref[...] = jnp.concatenate(vals, axis=1)
    idx_ref[...] = jnp.concatenate(idxs, axis=1)


def _tc3_call(qaT3, q2b, cr, cs2, ci):
    nsteps = NQ // QBLK3
    return pl.pallas_call(
        _tc3_body,
        grid=(nsteps,),
        in_specs=[
            pl.BlockSpec((1, CH, QBLK3), lambda i: (i, 0, 0)),
            pl.BlockSpec((QBLK3, 128), lambda i: (i, 0)),
            pl.BlockSpec((QBLK3 * NCAND, CH), lambda i: (i, 0)),
            pl.BlockSpec((QBLK3, NCAND), lambda i: (i, 0)),
            pl.BlockSpec((QBLK3, NCAND), lambda i: (i, 0)),
        ],
        out_specs=[
            pl.BlockSpec((QBLK3, K), lambda i: (i, 0)),
            pl.BlockSpec((QBLK3, K), lambda i: (i, 0)),
        ],
        out_shape=[
            jax.ShapeDtypeStruct((NQ, K), jnp.float32),
            jax.ShapeDtypeStruct((NQ, K), jnp.int32),
        ],
    )(qaT3, q2b, cr, cs2, ci)


def kernel(query, support):
    q = query[0]
    s = support[0]
    s_pad = jnp.concatenate(
        [s, jnp.full((NSP - NS, D), PADV, jnp.float32)], axis=0)
    s2 = jnp.sum(s_pad * s_pad, axis=-1)
    q2 = jnp.sum(q * q, axis=-1)
    qT = q.T
    q2r = q2.reshape(1, NQ)
    q2b = jnp.broadcast_to(q2[:, None], (NQ, 128))
    s2_3d = s2.reshape(NB, BLK, 1)
    sidxf = jnp.arange(NSP, dtype=jnp.float32)[:, None]
    s_aug = jnp.concatenate(
        [s_pad, s2[:, None], sidxf,
         jnp.zeros((NSP, CH - D - 2), jnp.float32)],
        axis=1)
    s_aug3 = s_aug.reshape(NG, (G * CH) // 128, 128)
    qaT = jnp.concatenate([qT, jnp.zeros((CH - D, NQ), jnp.float32)], axis=0)
    qaT3 = qaT.reshape(CH, NQ // QBLK3, QBLK3).swapaxes(0, 1)

    gidx_T = _tc1_call(qT, q2r, s_pad, s2_3d)
    gidx_flat = gidx_T.T.reshape(NQ * K)
    (rows,) = _sc_gather(gidx_flat, s_aug3)
    rows4 = rows.reshape(NQ, NCAND, CH)
    cr = rows.reshape(NQ * NCAND, CH)
    cs2 = rows4[:, :, D]
    ci = rows4[:, :, D + 1].astype(jnp.int32)
    vals, idx = _tc3_call(qaT3, q2b, cr, cs2, ci)
    return (vals[None], idx[None])
